# baseline (speedup 1.0000x reference)
import jax
import jax.numpy as jnp
from jax.experimental import pallas as pl
from jax.experimental.pallas import tpu as pltpu

_EPS = 1e-8
_BB = 8


def _hgp_kernel(aux_ref, inc_ref, nf_ref, ch_ref, neut_ref, em_ref, mask_ref,
                itnf_ref, nfs_ref):
    H = inc_ref.shape[1]
    N = inc_ref.shape[2]
    for i in range(_BB):
        inc = inc_ref[i]
        nf = nf_ref[i]
        a = aux_ref[i]

        ier = inc * a[0:1, :]
        s = ier.sum(axis=1, keepdims=True)
        denom = s + _EPS
        incn = ier / denom

        w4 = jnp.transpose(a[1:5, :], (1, 0))
        sums = jnp.dot(incn, w4,
                       preferred_element_type=jnp.float32)
        st = jnp.transpose(jnp.concatenate([sums, s], axis=1), (1, 0))

        ke = jnp.log1p(jnp.maximum(jnp.maximum(st[4:5, :], 0.0), 0.0))
        phi = jnp.arctan2(st[2:3, :], st[1:2, :])

        zpad = jnp.zeros((1, H - N), dtype=jnp.float32)
        tr = a[5:6, :]
        mask_h = jnp.concatenate([tr, zpad], axis=1)
        keep = jnp.where(mask_h > 0, 0.0, 1.0)

        ch3 = jnp.concatenate(
            [jnp.concatenate([a[6:7, :] * tr, zpad], axis=1),
             jnp.concatenate([a[7:8, :] * tr, zpad], axis=1),
             jnp.concatenate([a[8:9, :] * tr, zpad], axis=1)],
            axis=0,
        )
        neut3 = jnp.concatenate(
            [ke * keep, st[0:1, :] * keep, phi * keep], axis=0
        )

        ch_ref[i] = jnp.transpose(ch3, (1, 0))
        neut_ref[i] = jnp.transpose(neut3, (1, 0))
        em_ref[i:i + 1, :] = st[3:4, :]
        mask_ref[i:i + 1, :] = mask_h

        itnf_ref[i] = jnp.dot(inc, nf, preferred_element_type=jnp.float32)
        nfs_ref[i] = nf.sum(axis=0, keepdims=True)


def kernel(pred_inc, node_feat, e_raw, is_topo, is_track, track_pt, track_eta,
           track_phi, topo_eta_raw, topo_phi, topo_em_frac):
    B, H, N = pred_inc.shape
    D = node_feat.shape[2]

    topo_f = is_topo.astype(jnp.float32)
    zeros_bn = jnp.zeros_like(e_raw)
    aux = jnp.stack(
        [e_raw * topo_f,
         topo_eta_raw * topo_f,
         jnp.cos(topo_phi) * topo_f,
         jnp.sin(topo_phi) * topo_f,
         topo_em_frac,
         is_track.astype(jnp.float32),
         track_pt,
         track_eta,
         track_phi,
         zeros_bn, zeros_bn, zeros_bn, zeros_bn, zeros_bn, zeros_bn, zeros_bn],
        axis=1,
    )

    ch, neut, em, mask, itnf, nfs = pl.pallas_call(
        _hgp_kernel,
        grid=(B // _BB,),
        in_specs=[
            pl.BlockSpec((_BB, 16, N), lambda b: (b, 0, 0)),
            pl.BlockSpec((_BB, H, N), lambda b: (b, 0, 0)),
            pl.BlockSpec((_BB, N, D), lambda b: (b, 0, 0)),
        ],
        out_specs=[
            pl.BlockSpec((_BB, H, 3), lambda b: (b, 0, 0)),
            pl.BlockSpec((_BB, H, 3), lambda b: (b, 0, 0)),
            pl.BlockSpec((_BB, H), lambda b: (b, 0)),
            pl.BlockSpec((_BB, H), lambda b: (b, 0)),
            pl.BlockSpec((_BB, H, D), lambda b: (b, 0, 0)),
            pl.BlockSpec((_BB, 1, D), lambda b: (b, 0, 0)),
        ],
        out_shape=[
            jax.ShapeDtypeStruct((B, H, 3), jnp.float32),
            jax.ShapeDtypeStruct((B, H, 3), jnp.float32),
            jax.ShapeDtypeStruct((B, H), jnp.float32),
            jax.ShapeDtypeStruct((B, H), jnp.float32),
            jax.ShapeDtypeStruct((B, H, D), jnp.float32),
            jax.ShapeDtypeStruct((B, 1, D), jnp.float32),
        ],
        compiler_params=pltpu.CompilerParams(
            dimension_semantics=("arbitrary",),
        ),
    )(aux, pred_inc, node_feat)

    proxy_is_charged = mask > 0
    node_feat_sum = nfs[:, 0, :]
    return (ch, neut, proxy_is_charged, em, itnf, node_feat_sum)


# speedup vs baseline: 1.3247x; 1.3247x over previous
import jax
import jax.numpy as jnp
from jax.experimental import pallas as pl
from jax.experimental.pallas import tpu as pltpu
from jax.experimental.pallas import tpu_sc as plsc

_EPS = 1e-8
_BB = 8
_SC_LANES = 16


def _hgp_kernel(aux_ref, inc_ref, nf_ref, small_ref, itnf_ref, nfs_ref):
    H = inc_ref.shape[1]
    N = inc_ref.shape[2]
    for i in range(_BB):
        inc = inc_ref[i]
        nf = nf_ref[i]
        a = aux_ref[i]

        ier = inc * a[0:1, :]
        s = ier.sum(axis=1, keepdims=True)
        denom = s + _EPS
        incn = ier / denom

        w4 = jnp.transpose(a[1:5, :], (1, 0))
        sums = jnp.dot(incn, w4,
                       preferred_element_type=jnp.float32)
        st = jnp.transpose(jnp.concatenate([sums, s], axis=1), (1, 0))

        ke = jnp.log1p(jnp.maximum(jnp.maximum(st[4:5, :], 0.0), 0.0))
        phi = jnp.arctan2(st[2:3, :], st[1:2, :])

        zpad = jnp.zeros((1, H - N), dtype=jnp.float32)
        mask_h = jnp.concatenate([a[5:6, :], zpad], axis=1)
        keep = jnp.where(mask_h > 0, 0.0, 1.0)

        small_ref[i] = jnp.concatenate(
            [ke * keep, st[0:1, :] * keep, phi * keep, st[3:4, :]],
            axis=0,
        )

        itnf_ref[i] = jnp.dot(inc, nf, preferred_element_type=jnp.float32)
        nfs_ref[i] = nf.sum(axis=0, keepdims=True)


def _sc_charged(pt, eta, phi, mask, H):
    B, N = pt.shape

    @pl.kernel(
        out_type=jax.ShapeDtypeStruct((B, 4, H), jnp.float32),
        mesh=plsc.VectorSubcoreMesh(core_axis_name="core",
                                    subcore_axis_name="subcore"),
    )
    def sc_kernel(pt_hbm, eta_hbm, phi_hbm, m_hbm, o_hbm):
        def body(pt_vmem, eta_vmem, phi_vmem, m_vmem, o_vmem):
            o2 = o_vmem.at[0]
            rows = (pt_vmem, eta_vmem, phi_vmem, None)
            for r in range(4):
                @pl.loop(0, N, step=_SC_LANES)
                def _col(c, r=r):
                    src = (pl.ds(0, 1), pl.ds(c, _SC_LANES))
                    dst = (pl.ds(r, 1), pl.ds(c, _SC_LANES))
                    m = m_vmem.at[src][...]
                    if rows[r] is None:
                        o2.at[dst][...] = m
                    else:
                        o2.at[dst][...] = rows[r].at[src][...] * m

                @pl.loop(N, H, step=_SC_LANES)
                def _pad(c, r=r):
                    dst = (pl.ds(r, 1), pl.ds(c, _SC_LANES))
                    o2.at[dst][...] = jnp.zeros((1, _SC_LANES), jnp.float32)

        pltpu.emit_pipeline(
            body,
            grid=(B,),
            in_specs=[
                pl.BlockSpec((1, N), lambda i: (i, 0)),
                pl.BlockSpec((1, N), lambda i: (i, 0)),
                pl.BlockSpec((1, N), lambda i: (i, 0)),
                pl.BlockSpec((1, N), lambda i: (i, 0)),
            ],
            out_specs=[pl.BlockSpec((1, 4, H), lambda i: (i, 0, 0))],
            core_axis_name=("core", "subcore"),
            dimension_semantics=(pltpu.PARALLEL,),
        )(pt_hbm, eta_hbm, phi_hbm, m_hbm, o_hbm)

    return sc_kernel(pt, eta, phi, mask)


def kernel(pred_inc, node_feat, e_raw, is_topo, is_track, track_pt, track_eta,
           track_phi, topo_eta_raw, topo_phi, topo_em_frac):
    B, H, N = pred_inc.shape
    D = node_feat.shape[2]

    topo_f = is_topo.astype(jnp.float32)
    track_f = is_track.astype(jnp.float32)
    zeros_bn = jnp.zeros_like(e_raw)
    aux = jnp.stack(
        [e_raw * topo_f,
         topo_eta_raw * topo_f,
         jnp.cos(topo_phi) * topo_f,
         jnp.sin(topo_phi) * topo_f,
         topo_em_frac,
         track_f,
         zeros_bn, zeros_bn, zeros_bn, zeros_bn, zeros_bn, zeros_bn,
         zeros_bn, zeros_bn, zeros_bn, zeros_bn],
        axis=1,
    )

    sc_out = _sc_charged(track_pt, track_eta, track_phi, track_f, H)

    small, itnf, nfs = pl.pallas_call(
        _hgp_kernel,
        grid=(B // _BB,),
        in_specs=[
            pl.BlockSpec((_BB, 16, N), lambda b: (b, 0, 0)),
            pl.BlockSpec((_BB, H, N), lambda b: (b, 0, 0)),
            pl.BlockSpec((_BB, N, D), lambda b: (b, 0, 0)),
        ],
        out_specs=[
            pl.BlockSpec((_BB, 4, H), lambda b: (b, 0, 0)),
            pl.BlockSpec((_BB, H, D), lambda b: (b, 0, 0)),
            pl.BlockSpec((_BB, 1, D), lambda b: (b, 0, 0)),
        ],
        out_shape=[
            jax.ShapeDtypeStruct((B, 4, H), jnp.float32),
            jax.ShapeDtypeStruct((B, H, D), jnp.float32),
            jax.ShapeDtypeStruct((B, 1, D), jnp.float32),
        ],
        compiler_params=pltpu.CompilerParams(
            dimension_semantics=("arbitrary",),
        ),
    )(aux, pred_inc, node_feat)

    charged_proxy_kin = jnp.transpose(sc_out[:, 0:3, :], (0, 2, 1))
    proxy_is_charged = sc_out[:, 3, :] > 0
    neut_proxy_kin = jnp.transpose(small[:, 0:3, :], (0, 2, 1))
    proxy_em_frac = small[:, 3, :]
    node_feat_sum = nfs[:, 0, :]
    return (charged_proxy_kin, neut_proxy_kin, proxy_is_charged, proxy_em_frac,
            itnf, node_feat_sum)


# speedup vs baseline: 1.3295x; 1.0036x over previous
import jax
import jax.numpy as jnp
from jax.experimental import pallas as pl
from jax.experimental.pallas import tpu as pltpu
from jax.experimental.pallas import tpu_sc as plsc

_EPS = 1e-8
_BB = 8
_SC_LANES = 16


def _hgp_kernel(aux_ref, inc_ref, nf_ref, small_ref, itnf_ref, nfs_ref):
    H = inc_ref.shape[1]
    N = inc_ref.shape[2]
    for i in range(_BB):
        inc = inc_ref[i]
        nf = nf_ref[i]
        a = aux_ref[i]

        ier = inc * a[0:1, :]
        s = ier.sum(axis=1, keepdims=True)
        denom = s + _EPS
        incn = ier / denom

        w4 = jnp.transpose(a[1:5, :], (1, 0))
        sums = jnp.dot(incn, w4,
                       preferred_element_type=jnp.float32)
        st = jnp.transpose(jnp.concatenate([sums, s], axis=1), (1, 0))

        ke = jnp.log1p(jnp.maximum(jnp.maximum(st[4:5, :], 0.0), 0.0))
        phi = jnp.arctan2(st[2:3, :], st[1:2, :])

        zpad = jnp.zeros((1, H - N), dtype=jnp.float32)
        mask_h = jnp.concatenate([a[5:6, :], zpad], axis=1)
        keep = jnp.where(mask_h > 0, 0.0, 1.0)

        small_ref[i] = jnp.concatenate(
            [ke * keep, st[0:1, :] * keep, phi * keep, st[3:4, :]],
            axis=0,
        )

        itnf_ref[i] = jnp.dot(inc, nf, preferred_element_type=jnp.float32)
        nfs_ref[i] = nf.sum(axis=0, keepdims=True)


def _sc_charged(pt, eta, phi, mask, H):
    B, N = pt.shape

    @pl.kernel(
        out_type=jax.ShapeDtypeStruct((B, 4, H), jnp.float32),
        mesh=plsc.VectorSubcoreMesh(core_axis_name="core",
                                    subcore_axis_name="subcore"),
    )
    def sc_kernel(pt_hbm, eta_hbm, phi_hbm, m_hbm, o_hbm):
        def body(pt_vmem, eta_vmem, phi_vmem, m_vmem, o_vmem):
            o2 = o_vmem.at[0]
            rows = (pt_vmem, eta_vmem, phi_vmem, None)
            for r in range(4):
                @pl.loop(0, N, step=_SC_LANES)
                def _col(c, r=r):
                    src = (pl.ds(0, 1), pl.ds(c, _SC_LANES))
                    dst = (pl.ds(r, 1), pl.ds(c, _SC_LANES))
                    m = m_vmem.at[src][...]
                    if rows[r] is None:
                        o2.at[dst][...] = m
                    else:
                        o2.at[dst][...] = rows[r].at[src][...] * m

                @pl.loop(N, H, step=_SC_LANES)
                def _pad(c, r=r):
                    dst = (pl.ds(r, 1), pl.ds(c, _SC_LANES))
                    o2.at[dst][...] = jnp.zeros((1, _SC_LANES), jnp.float32)

        pltpu.emit_pipeline(
            body,
            grid=(B,),
            in_specs=[
                pl.BlockSpec((1, N), lambda i: (i, 0)),
                pl.BlockSpec((1, N), lambda i: (i, 0)),
                pl.BlockSpec((1, N), lambda i: (i, 0)),
                pl.BlockSpec((1, N), lambda i: (i, 0)),
            ],
            out_specs=[pl.BlockSpec((1, 4, H), lambda i: (i, 0, 0))],
            core_axis_name=("core", "subcore"),
            dimension_semantics=(pltpu.PARALLEL,),
        )(pt_hbm, eta_hbm, phi_hbm, m_hbm, o_hbm)

    return sc_kernel(pt, eta, phi, mask)


def kernel(pred_inc, node_feat, e_raw, is_topo, is_track, track_pt, track_eta,
           track_phi, topo_eta_raw, topo_phi, topo_em_frac):
    B, H, N = pred_inc.shape
    D = node_feat.shape[2]

    topo_f = is_topo.astype(jnp.float32)
    track_f = is_track.astype(jnp.float32)
    zeros_bn = jnp.zeros_like(e_raw)
    aux = jnp.stack(
        [e_raw * topo_f,
         topo_eta_raw * topo_f,
         jnp.cos(topo_phi) * topo_f,
         jnp.sin(topo_phi) * topo_f,
         topo_em_frac,
         track_f,
         zeros_bn, zeros_bn, zeros_bn, zeros_bn, zeros_bn, zeros_bn,
         zeros_bn, zeros_bn, zeros_bn, zeros_bn],
        axis=1,
    )

    small, itnf, nfs = pl.pallas_call(
        _hgp_kernel,
        grid=(B // _BB,),
        in_specs=[
            pl.BlockSpec((_BB, 16, N), lambda b: (b, 0, 0)),
            pl.BlockSpec((_BB, H, N), lambda b: (b, 0, 0)),
            pl.BlockSpec((_BB, N, D), lambda b: (b, 0, 0)),
        ],
        out_specs=[
            pl.BlockSpec((_BB, 4, H), lambda b: (b, 0, 0)),
            pl.BlockSpec((_BB, H, D), lambda b: (b, 0, 0)),
            pl.BlockSpec((_BB, 1, D), lambda b: (b, 0, 0)),
        ],
        out_shape=[
            jax.ShapeDtypeStruct((B, 4, H), jnp.float32),
            jax.ShapeDtypeStruct((B, H, D), jnp.float32),
            jax.ShapeDtypeStruct((B, 1, D), jnp.float32),
        ],
        compiler_params=pltpu.CompilerParams(
            dimension_semantics=("arbitrary",),
        ),
    )(aux, pred_inc, node_feat)

    sc_out = _sc_charged(track_pt, track_eta, track_phi, track_f, H)

    charged_proxy_kin = jnp.transpose(sc_out[:, 0:3, :], (0, 2, 1))
    proxy_is_charged = sc_out[:, 3, :] > 0
    neut_proxy_kin = jnp.transpose(small[:, 0:3, :], (0, 2, 1))
    proxy_em_frac = small[:, 3, :]
    node_feat_sum = nfs[:, 0, :]
    return (charged_proxy_kin, neut_proxy_kin, proxy_is_charged, proxy_em_frac,
            itnf, node_feat_sum)


# speedup vs baseline: 1.3852x; 1.0419x over previous
import jax
import jax.numpy as jnp
from jax.experimental import pallas as pl
from jax.experimental.pallas import tpu as pltpu
from jax.experimental.pallas import tpu_sc as plsc

_EPS = 1e-8
_BB = 16
_SC_LANES = 16


def _hgp_kernel(aux_ref, inc_ref, nf_ref, small_ref, itnf_ref, nfs_ref):
    H = inc_ref.shape[1]
    N = inc_ref.shape[2]
    for i in range(_BB):
        inc = inc_ref[i]
        nf = nf_ref[i]
        a = aux_ref[i]

        ier = inc * a[0:1, :]
        s = ier.sum(axis=1, keepdims=True)
        denom = s + _EPS
        incn = ier / denom

        w4 = jnp.transpose(a[1:5, :], (1, 0))
        sums = jnp.dot(incn, w4,
                       preferred_element_type=jnp.float32)
        st = jnp.transpose(jnp.concatenate([sums, s], axis=1), (1, 0))

        ke = jnp.log1p(jnp.maximum(jnp.maximum(st[4:5, :], 0.0), 0.0))
        phi = jnp.arctan2(st[2:3, :], st[1:2, :])

        zpad = jnp.zeros((1, H - N), dtype=jnp.float32)
        mask_h = jnp.concatenate([a[5:6, :], zpad], axis=1)
        keep = jnp.where(mask_h > 0, 0.0, 1.0)

        small_ref[i] = jnp.concatenate(
            [ke * keep, st[0:1, :] * keep, phi * keep, st[3:4, :]],
            axis=0,
        )

        itnf_ref[i] = jnp.dot(inc, nf, preferred_element_type=jnp.float32)
        nfs_ref[i] = nf.sum(axis=0, keepdims=True)


def _sc_charged(pt, eta, phi, mask, H):
    B, N = pt.shape

    @pl.kernel(
        out_type=jax.ShapeDtypeStruct((B, 4, H), jnp.float32),
        mesh=plsc.VectorSubcoreMesh(core_axis_name="core",
                                    subcore_axis_name="subcore"),
    )
    def sc_kernel(pt_hbm, eta_hbm, phi_hbm, m_hbm, o_hbm):
        def body(pt_vmem, eta_vmem, phi_vmem, m_vmem, o_vmem):
            o2 = o_vmem.at[0]
            rows = (pt_vmem, eta_vmem, phi_vmem, None)
            for r in range(4):
                @pl.loop(0, N, step=_SC_LANES)
                def _col(c, r=r):
                    src = (pl.ds(0, 1), pl.ds(c, _SC_LANES))
                    dst = (pl.ds(r, 1), pl.ds(c, _SC_LANES))
                    m = m_vmem.at[src][...]
                    if rows[r] is None:
                        o2.at[dst][...] = m
                    else:
                        o2.at[dst][...] = rows[r].at[src][...] * m

                @pl.loop(N, H, step=_SC_LANES)
                def _pad(c, r=r):
                    dst = (pl.ds(r, 1), pl.ds(c, _SC_LANES))
                    o2.at[dst][...] = jnp.zeros((1, _SC_LANES), jnp.float32)

        pltpu.emit_pipeline(
            body,
            grid=(B,),
            in_specs=[
                pl.BlockSpec((1, N), lambda i: (i, 0)),
                pl.BlockSpec((1, N), lambda i: (i, 0)),
                pl.BlockSpec((1, N), lambda i: (i, 0)),
                pl.BlockSpec((1, N), lambda i: (i, 0)),
            ],
            out_specs=[pl.BlockSpec((1, 4, H), lambda i: (i, 0, 0))],
            core_axis_name=("core", "subcore"),
            dimension_semantics=(pltpu.PARALLEL,),
        )(pt_hbm, eta_hbm, phi_hbm, m_hbm, o_hbm)

    return sc_kernel(pt, eta, phi, mask)


def kernel(pred_inc, node_feat, e_raw, is_topo, is_track, track_pt, track_eta,
           track_phi, topo_eta_raw, topo_phi, topo_em_frac):
    B, H, N = pred_inc.shape
    D = node_feat.shape[2]

    topo_f = is_topo.astype(jnp.float32)
    track_f = is_track.astype(jnp.float32)
    zeros_bn = jnp.zeros_like(e_raw)
    aux = jnp.stack(
        [e_raw * topo_f,
         topo_eta_raw * topo_f,
         jnp.cos(topo_phi) * topo_f,
         jnp.sin(topo_phi) * topo_f,
         topo_em_frac,
         track_f,
         zeros_bn, zeros_bn, zeros_bn, zeros_bn, zeros_bn, zeros_bn,
         zeros_bn, zeros_bn, zeros_bn, zeros_bn],
        axis=1,
    )

    small, itnf, nfs = pl.pallas_call(
        _hgp_kernel,
        grid=(B // _BB,),
        in_specs=[
            pl.BlockSpec((_BB, 16, N), lambda b: (b, 0, 0)),
            pl.BlockSpec((_BB, H, N), lambda b: (b, 0, 0)),
            pl.BlockSpec((_BB, N, D), lambda b: (b, 0, 0)),
        ],
        out_specs=[
            pl.BlockSpec((_BB, 4, H), lambda b: (b, 0, 0)),
            pl.BlockSpec((_BB, H, D), lambda b: (b, 0, 0)),
            pl.BlockSpec((_BB, 1, D), lambda b: (b, 0, 0)),
        ],
        out_shape=[
            jax.ShapeDtypeStruct((B, 4, H), jnp.float32),
            jax.ShapeDtypeStruct((B, H, D), jnp.float32),
            jax.ShapeDtypeStruct((B, 1, D), jnp.float32),
        ],
        compiler_params=pltpu.CompilerParams(
            dimension_semantics=("arbitrary",),
        ),
    )(aux, pred_inc, node_feat)

    sc_out = _sc_charged(track_pt, track_eta, track_phi, track_f, H)

    charged_proxy_kin = jnp.transpose(sc_out[:, 0:3, :], (0, 2, 1))
    proxy_is_charged = sc_out[:, 3, :] > 0
    neut_proxy_kin = jnp.transpose(small[:, 0:3, :], (0, 2, 1))
    proxy_em_frac = small[:, 3, :]
    node_feat_sum = nfs[:, 0, :]
    return (charged_proxy_kin, neut_proxy_kin, proxy_is_charged, proxy_em_frac,
            itnf, node_feat_sum)


# speedup vs baseline: 1.4106x; 1.0183x over previous
import jax
import jax.numpy as jnp
from jax.experimental import pallas as pl
from jax.experimental.pallas import tpu as pltpu
from jax.experimental.pallas import tpu_sc as plsc

_EPS = 1e-8
_BB = 32
_SC_LANES = 16


def _hgp_kernel(aux_ref, inc_ref, nf_ref, small_ref, itnf_ref, nfs_ref):
    H = inc_ref.shape[1]
    N = inc_ref.shape[2]
    for i in range(_BB):
        inc = inc_ref[i]
        nf = nf_ref[i]
        a = aux_ref[i]

        ier = inc * a[0:1, :]
        s = ier.sum(axis=1, keepdims=True)
        denom = s + _EPS
        incn = ier / denom

        w4 = jnp.transpose(a[1:5, :], (1, 0))
        sums = jnp.dot(incn, w4,
                       preferred_element_type=jnp.float32)
        st = jnp.transpose(jnp.concatenate([sums, s], axis=1), (1, 0))

        ke = jnp.log1p(jnp.maximum(jnp.maximum(st[4:5, :], 0.0), 0.0))
        phi = jnp.arctan2(st[2:3, :], st[1:2, :])

        zpad = jnp.zeros((1, H - N), dtype=jnp.float32)
        mask_h = jnp.concatenate([a[5:6, :], zpad], axis=1)
        keep = jnp.where(mask_h > 0, 0.0, 1.0)

        small_ref[i] = jnp.concatenate(
            [ke * keep, st[0:1, :] * keep, phi * keep, st[3:4, :]],
            axis=0,
        )

        itnf_ref[i] = jnp.dot(inc, nf, preferred_element_type=jnp.float32)
        nfs_ref[i] = nf.sum(axis=0, keepdims=True)


def _sc_charged(pt, eta, phi, mask, H):
    B, N = pt.shape

    @pl.kernel(
        out_type=jax.ShapeDtypeStruct((B, 4, H), jnp.float32),
        mesh=plsc.VectorSubcoreMesh(core_axis_name="core",
                                    subcore_axis_name="subcore"),
    )
    def sc_kernel(pt_hbm, eta_hbm, phi_hbm, m_hbm, o_hbm):
        def body(pt_vmem, eta_vmem, phi_vmem, m_vmem, o_vmem):
            o2 = o_vmem.at[0]
            rows = (pt_vmem, eta_vmem, phi_vmem, None)
            for r in range(4):
                @pl.loop(0, N, step=_SC_LANES)
                def _col(c, r=r):
                    src = (pl.ds(0, 1), pl.ds(c, _SC_LANES))
                    dst = (pl.ds(r, 1), pl.ds(c, _SC_LANES))
                    m = m_vmem.at[src][...]
                    if rows[r] is None:
                        o2.at[dst][...] = m
                    else:
                        o2.at[dst][...] = rows[r].at[src][...] * m

                @pl.loop(N, H, step=_SC_LANES)
                def _pad(c, r=r):
                    dst = (pl.ds(r, 1), pl.ds(c, _SC_LANES))
                    o2.at[dst][...] = jnp.zeros((1, _SC_LANES), jnp.float32)

        pltpu.emit_pipeline(
            body,
            grid=(B,),
            in_specs=[
                pl.BlockSpec((1, N), lambda i: (i, 0)),
                pl.BlockSpec((1, N), lambda i: (i, 0)),
                pl.BlockSpec((1, N), lambda i: (i, 0)),
                pl.BlockSpec((1, N), lambda i: (i, 0)),
            ],
            out_specs=[pl.BlockSpec((1, 4, H), lambda i: (i, 0, 0))],
            core_axis_name=("core", "subcore"),
            dimension_semantics=(pltpu.PARALLEL,),
        )(pt_hbm, eta_hbm, phi_hbm, m_hbm, o_hbm)

    return sc_kernel(pt, eta, phi, mask)


def kernel(pred_inc, node_feat, e_raw, is_topo, is_track, track_pt, track_eta,
           track_phi, topo_eta_raw, topo_phi, topo_em_frac):
    B, H, N = pred_inc.shape
    D = node_feat.shape[2]

    topo_f = is_topo.astype(jnp.float32)
    track_f = is_track.astype(jnp.float32)
    zeros_bn = jnp.zeros_like(e_raw)
    aux = jnp.stack(
        [e_raw * topo_f,
         topo_eta_raw * topo_f,
         jnp.cos(topo_phi) * topo_f,
         jnp.sin(topo_phi) * topo_f,
         topo_em_frac,
         track_f,
         zeros_bn, zeros_bn, zeros_bn, zeros_bn, zeros_bn, zeros_bn,
         zeros_bn, zeros_bn, zeros_bn, zeros_bn],
        axis=1,
    )

    small, itnf, nfs = pl.pallas_call(
        _hgp_kernel,
        grid=(B // _BB,),
        in_specs=[
            pl.BlockSpec((_BB, 16, N), lambda b: (b, 0, 0)),
            pl.BlockSpec((_BB, H, N), lambda b: (b, 0, 0)),
            pl.BlockSpec((_BB, N, D), lambda b: (b, 0, 0)),
        ],
        out_specs=[
            pl.BlockSpec((_BB, 4, H), lambda b: (b, 0, 0)),
            pl.BlockSpec((_BB, H, D), lambda b: (b, 0, 0)),
            pl.BlockSpec((_BB, 1, D), lambda b: (b, 0, 0)),
        ],
        out_shape=[
            jax.ShapeDtypeStruct((B, 4, H), jnp.float32),
            jax.ShapeDtypeStruct((B, H, D), jnp.float32),
            jax.ShapeDtypeStruct((B, 1, D), jnp.float32),
        ],
        compiler_params=pltpu.CompilerParams(
            dimension_semantics=("arbitrary",),
        ),
    )(aux, pred_inc, node_feat)

    sc_out = _sc_charged(track_pt, track_eta, track_phi, track_f, H)

    charged_proxy_kin = jnp.transpose(sc_out[:, 0:3, :], (0, 2, 1))
    proxy_is_charged = sc_out[:, 3, :] > 0
    neut_proxy_kin = jnp.transpose(small[:, 0:3, :], (0, 2, 1))
    proxy_em_frac = small[:, 3, :]
    node_feat_sum = nfs[:, 0, :]
    return (charged_proxy_kin, neut_proxy_kin, proxy_is_charged, proxy_em_frac,
            itnf, node_feat_sum)
